# baseline (speedup 1.0000x reference)
import functools

import jax
import jax.numpy as jnp
from jax import lax
from jax.experimental import pallas as pl
from jax.experimental.pallas import tpu as pltpu
from jax.experimental.pallas import tpu_sc as plsc

D = 128
NB = 4
NBUF = 4


@jax.jit
def _embed(idx, table):
    B, H = idx.shape

    mesh = plsc.VectorSubcoreMesh(core_axis_name="c", subcore_axis_name="s")
    info = plsc.get_sparse_core_info()
    NC = info.num_cores
    NW = NC * info.num_subcores
    b_per_w = B // NW
    n_chunks = b_per_w // NB

    @functools.partial(
        pl.kernel,
        out_type=jax.ShapeDtypeStruct((B, H, D), jnp.float32),
        mesh=mesh,
        compiler_params=pltpu.CompilerParams(use_tc_tiling_on_sc=True),
        scratch_types=(
            [pltpu.VMEM((b_per_w, H), jnp.int32)]
            + [pltpu.VMEM((NB, H, D), jnp.float32) for _ in range(NBUF)]
            + [pltpu.SemaphoreType.DMA for _ in range(2 * NBUF)]
        ),
    )
    def body(idx_hbm, table_hbm, out_hbm, idx_v, *rest):
        bufs = rest[:NBUF]
        gsem = rest[NBUF:2 * NBUF]
        osem = rest[2 * NBUF:]
        wid = lax.axis_index("s") * NC + lax.axis_index("c")
        base = wid * b_per_w
        pltpu.sync_copy(idx_hbm.at[pl.ds(base, b_per_w)], idx_v)

        def start_gathers(c, r):
            for k in range(NB):
                pltpu.async_copy(
                    table_hbm.at[idx_v.at[c * NB + k]],
                    bufs[r].at[k], gsem[r])

        def wait_gathers(c, r):
            for k in range(NB):
                pltpu.make_async_copy(
                    table_hbm.at[idx_v.at[c * NB + k]],
                    bufs[r].at[k], gsem[r]).wait()

        def start_out(c, r):
            pltpu.async_copy(
                bufs[r], out_hbm.at[pl.ds(base + c * NB, NB)], osem[r])

        def wait_out(c, r):
            pltpu.make_async_copy(
                bufs[r], out_hbm.at[pl.ds(base + c * NB, NB)], osem[r]).wait()

        for r in range(NBUF):
            start_gathers(r, r)

        @pl.loop(0, n_chunks - NBUF, step=NBUF)
        def _steady(c0):
            for r in range(NBUF):
                c = c0 + r
                wait_gathers(c, r)
                start_out(c, r)
                wait_out(c, r)
                start_gathers(c + NBUF, r)

        for r in range(NBUF):
            c = n_chunks - NBUF + r
            wait_gathers(c, r)
            start_out(c, r)
            wait_out(c, r)

    return body(idx, table)


def kernel(input, table):
    return _embed(input.astype(jnp.int32), table)


# speedup vs baseline: 1.0078x; 1.0078x over previous
import functools

import jax
import jax.numpy as jnp
from jax import lax
from jax.experimental import pallas as pl
from jax.experimental.pallas import tpu as pltpu
from jax.experimental.pallas import tpu_sc as plsc

D = 128
NB = 2
NBUF = 8
W = 4


@jax.jit
def _embed(idx, table):
    B, H = idx.shape

    mesh = plsc.VectorSubcoreMesh(core_axis_name="c", subcore_axis_name="s")
    info = plsc.get_sparse_core_info()
    NC = info.num_cores
    NW = NC * info.num_subcores
    b_per_w = B // NW
    n_chunks = b_per_w // NB

    @functools.partial(
        pl.kernel,
        out_type=jax.ShapeDtypeStruct((B, H, D), jnp.float32),
        mesh=mesh,
        compiler_params=pltpu.CompilerParams(use_tc_tiling_on_sc=True),
        scratch_types=(
            [pltpu.VMEM((b_per_w, H), jnp.int32)]
            + [pltpu.VMEM((NB, H, D), jnp.float32) for _ in range(NBUF)]
            + [pltpu.SemaphoreType.DMA for _ in range(2 * NBUF)]
        ),
    )
    def body(idx_hbm, table_hbm, out_hbm, idx_v, *rest):
        bufs = rest[:NBUF]
        gsem = rest[NBUF:2 * NBUF]
        osem = rest[2 * NBUF:]
        wid = lax.axis_index("s") * NC + lax.axis_index("c")
        base = wid * b_per_w
        pltpu.sync_copy(idx_hbm.at[pl.ds(base, b_per_w)], idx_v)

        def start_gathers(c, r):
            for k in range(NB):
                pltpu.async_copy(
                    table_hbm.at[idx_v.at[c * NB + k]],
                    bufs[r].at[k], gsem[r])

        def wait_gathers(c, r):
            for k in range(NB):
                pltpu.make_async_copy(
                    table_hbm.at[idx_v.at[c * NB + k]],
                    bufs[r].at[k], gsem[r]).wait()

        def start_out(c, r):
            pltpu.async_copy(
                bufs[r], out_hbm.at[pl.ds(base + c * NB, NB)], osem[r])

        def wait_out(c, r):
            pltpu.make_async_copy(
                bufs[r], out_hbm.at[pl.ds(base + c * NB, NB)], osem[r]).wait()

        for r in range(NBUF):
            start_gathers(r, r)

        for c in range(W):
            wait_gathers(c, c % NBUF)
            start_out(c, c % NBUF)

        @pl.loop(W, n_chunks - NBUF + W, step=NBUF)
        def _steady(c0):
            for r0 in range(NBUF):
                c = c0 + r0
                rc = (W + r0) % NBUF
                wait_gathers(c, rc)
                start_out(c, rc)
                co = c - W
                wait_out(co, r0)
                start_gathers(co + NBUF, r0)

        for c in range(n_chunks - NBUF + W, n_chunks):
            r = c % NBUF
            wait_gathers(c, r)
            start_out(c, r)
            co = c - W
            wait_out(co, co % NBUF)

        for co in range(n_chunks - W, n_chunks):
            wait_out(co, co % NBUF)

    return body(idx, table)


def kernel(input, table):
    return _embed(input.astype(jnp.int32), table)
